# Initial kernel scaffold; baseline (speedup 1.0000x reference)
import jax
import jax.numpy as jnp
from jax.experimental import pallas as pl


def kernel(src, tgt):
    raise NotImplementedError("write your pallas kernel here")


# speedup vs baseline: 47.9545x; 47.9545x over previous
import functools

import jax
import jax.numpy as jnp
import numpy as np
from jax import lax
from jax.experimental import pallas as pl
from jax.experimental.pallas import tpu as pltpu
from jax.experimental.pallas import tpu_sc as plsc

NBINS = 256
H = 512
W = 512
C = 3
NPIX = H * W
NTOT = H * W * C
NC = 2
NS = 16
NW = NC * NS
LANES = 16
CHUNK = NTOT // NW
GROUPS = CHUNK // (3 * LANES)
NHIST = 2 * C * NBINS
TROWS = 8
TCOLS = C * NBINS
SENTINEL = -3.0e38

def _lane_iota():
    return lax.broadcasted_iota(jnp.int32, (LANES,), 0)


def _bin_index(v):
    vc = jnp.minimum(jnp.maximum(v, -1.0), 1.0)
    t = (vc + 1.0) * 128.0
    return jnp.minimum(t.astype(jnp.int32), NBINS - 1)


def _hist_body(src_hbm, tgt_hbm, out_hbm, xbuf, priv, red):
    wid = lax.axis_index("s") * NC + lax.axis_index("c")
    base = wid * CHUNK

    lane = _lane_iota()
    zeros16 = lane * 0
    ones16 = zeros16 + 1
    hoff = [lane * NHIST + lax.rem(lane + p, 3) * NBINS for p in range(3)]

    def zero_body(i, _):
        priv[pl.ds(i * LANES, LANES)] = zeros16
        return _

    lax.fori_loop(0, (LANES * NHIST) // LANES, zero_body, None)

    for img, inp in ((0, src_hbm), (1, tgt_hbm)):
        pltpu.sync_copy(inp.at[pl.ds(base, CHUNK)], xbuf)
        offs = [hoff[p] + img * C * NBINS for p in range(3)]

        def gbody(g, _, offs=offs):
            for p in range(3):
                v = xbuf[pl.ds(g * (3 * LANES) + p * LANES, LANES)]
                idx = _bin_index(v) + offs[p]
                plsc.addupdate_scatter(priv, [idx], ones16)
            return _

        lax.fori_loop(0, GROUPS, gbody, None)

    def rbody(k, _):
        acc = priv[pl.ds(k * LANES, LANES)]
        for l in range(1, LANES):
            acc = acc + priv[pl.ds(l * NHIST + k * LANES, LANES)]
        red[pl.ds(k * LANES, LANES)] = acc
        return _

    lax.fori_loop(0, NHIST // LANES, rbody, None)
    pltpu.sync_copy(red, out_hbm.at[pl.ds(wid * NHIST, NHIST)])


def _sc_hist(src_f, tgt_f):
    mesh = plsc.VectorSubcoreMesh(
        core_axis_name="c", subcore_axis_name="s", num_cores=NC,
        num_subcores=NS)
    return pl.kernel(
        _hist_body,
        out_type=jax.ShapeDtypeStruct((NW * NHIST,), jnp.int32),
        mesh=mesh,
        compiler_params=pltpu.CompilerParams(needs_layout_passes=False),
        scratch_types=[
            pltpu.VMEM((CHUNK,), jnp.float32),
            pltpu.VMEM((LANES * NHIST,), jnp.int32),
            pltpu.VMEM((NHIST,), jnp.int32),
        ],
    )(src_f, tgt_f)


def _gather_row(tab_row, ind_col, n):
    cols = lax.broadcasted_iota(jnp.int32, (NBINS, n), 1)
    sel = jnp.where(cols == ind_col, jnp.broadcast_to(tab_row, (NBINS, n)),
                    0.0)
    return jnp.sum(sel, axis=1, keepdims=True)


def _transpose_col(col):
    rows = lax.broadcasted_iota(jnp.int32, (NBINS, NBINS), 0)
    cols = lax.broadcasted_iota(jnp.int32, (NBINS, NBINS), 1)
    sel = jnp.where(rows == cols, jnp.broadcast_to(col, (NBINS, NBINS)), 0.0)
    return jnp.sum(sel, axis=0, keepdims=True)


def _argmin_first(absd):
    n = absd.shape[1]
    mn = jnp.min(absd, axis=1, keepdims=True)
    cols = lax.broadcasted_iota(jnp.int32, (NBINS, n), 1)
    return jnp.min(jnp.where(absd == mn, cols, n), axis=1, keepdims=True)


def _interp_ref(dx_row, dy_row, xs_col, ny):
    absd = jnp.abs(jnp.broadcast_to(dx_row, (NBINS, NBINS)) - xs_col)
    ind1 = _argmin_first(absd)
    ind0 = ind1 - 1
    ind0w256 = ind0 + jnp.where(ind0 < 0, NBINS, 0)
    ind0wny = ind0 + jnp.where(ind0 < 0, ny, 0)
    dx0 = _gather_row(dx_row, ind0w256, NBINS)
    dx1 = _gather_row(dx_row, ind1, NBINS)
    dy0 = _gather_row(dy_row, ind0wny, ny)
    dy1 = _gather_row(dy_row, ind1, ny)
    interp = dy0 + (dy1 - dy0) * (xs_col - dx0) / (dx1 - dx0)
    lo = dx_row[0, 0]
    hi = dx_row[0, NBINS - 1]
    return jnp.where(xs_col <= lo, dy_row[0, 0],
                     jnp.where(xs_col >= hi, dy_row[0, ny - 1], interp))


def _tc_tables_body(h_ref, t_ref):
    hs = jnp.sum(h_ref[...], axis=0, keepdims=True)
    gridk = lax.broadcasted_iota(jnp.int32, (1, 2 * NBINS), 1)
    grid = gridk.astype(jnp.float32) * (1.0 / 256.0) - 1.0
    scale = jnp.float32(2.0)
    denom = jnp.float32(NPIX - 1)

    rows_m, rows_a, rows_b = [], [], []
    rows_tlo, rows_thi, rows_vlo, rows_vhi = [], [], [], []
    for c in range(C):
        def cdf_of(off):
            hist = hs[:, off:off + NBINS]
            cdf = hist
            for k in (1, 2, 4, 8, 16, 32, 64, 128):
                cdf = cdf + jnp.concatenate(
                    [jnp.zeros((1, k), cdf.dtype), cdf[:, :-k]], axis=1)
            cdfmin = jnp.min(cdf)
            return (cdf - cdfmin).astype(jnp.float32) * scale / denom - 1.0

        cs_row = cdf_of(c * NBINS)
        ct_row = cdf_of(C * NBINS + c * NBINS)
        cs_col = jnp.sum(
            jnp.where(
                lax.broadcasted_iota(jnp.int32, (NBINS, NBINS), 0)
                == lax.broadcasted_iota(jnp.int32, (NBINS, NBINS), 1),
                jnp.broadcast_to(cs_row, (NBINS, NBINS)), 0.0),
            axis=1, keepdims=True)

        pm_col = _interp_ref(ct_row, grid, cs_col, 2 * NBINS)
        pm_row = _transpose_col(pm_col)

        eq = (jnp.broadcast_to(cs_row, (NBINS, NBINS)) == cs_col)
        cols = lax.broadcasted_iota(jnp.int32, (NBINS, NBINS), 1)
        f_col = jnp.min(jnp.where(eq, cols, NBINS), axis=1, keepdims=True)

        ind1 = f_col
        ind0 = ind1 - 1
        ind0w = ind0 + jnp.where(ind0 < 0, NBINS, 0)
        dx0 = _gather_row(cs_row, ind0w, NBINS)
        dx1 = _gather_row(cs_row, ind1, NBINS)
        dy0 = _gather_row(pm_row, ind0w, NBINS)
        dy1 = _gather_row(pm_row, ind1, NBINS)
        b_col = (dy1 - dy0) / (dx1 - dx0)
        a_col = dy0 - b_col * dx0

        m_col = jnp.concatenate(
            [jnp.full((1, 1), SENTINEL, jnp.float32),
             (cs_col[:-1, :] + cs_col[1:, :]) * 0.5], axis=0)

        rows_m.append(_transpose_col(m_col))
        rows_a.append(_transpose_col(a_col))
        rows_b.append(_transpose_col(b_col))
        rows_tlo.append(jnp.full((1, NBINS), cs_row[0, 0], jnp.float32))
        rows_thi.append(jnp.full((1, NBINS), cs_row[0, NBINS - 1],
                                 jnp.float32))
        rows_vlo.append(jnp.full((1, NBINS), pm_col[0, 0], jnp.float32))
        rows_vhi.append(jnp.full((1, NBINS), pm_col[NBINS - 1, 0],
                                 jnp.float32))

    t_ref[...] = jnp.concatenate(
        [jnp.concatenate(rows_m, axis=1),
         jnp.concatenate(rows_a, axis=1),
         jnp.concatenate(rows_b, axis=1),
         jnp.concatenate(rows_tlo, axis=1),
         jnp.concatenate(rows_thi, axis=1),
         jnp.concatenate(rows_vlo, axis=1),
         jnp.concatenate(rows_vhi, axis=1),
         jnp.zeros((1, TCOLS), jnp.float32)], axis=0)


def _tc_tables(hparts):
    return pl.pallas_call(
        _tc_tables_body,
        out_shape=jax.ShapeDtypeStruct((TROWS, TCOLS), jnp.float32),
    )(hparts)


def _map_body(src_hbm, tab_hbm, out_hbm, xbuf, tbuf):
    wid = lax.axis_index("s") * NC + lax.axis_index("c")
    base = wid * CHUNK
    pltpu.sync_copy(tab_hbm, tbuf)
    pltpu.sync_copy(src_hbm.at[pl.ds(base, CHUNK)], xbuf)

    lane = _lane_iota()
    ch256 = [lax.rem(lane + p, 3) * NBINS for p in range(3)]

    def gbody(g, _):
        for p in range(3):
            s0 = g * (3 * LANES) + p * LANES
            x = xbuf[pl.ds(s0, LANES)]
            r = ch256[p]
            for step in (128, 64, 32, 16, 8, 4, 2, 1):
                probe = r + step
                mv = plsc.load_gather(tbuf, [probe])
                r = jnp.where(mv < x, probe, r)
            a = plsc.load_gather(tbuf, [r + TCOLS])
            b = plsc.load_gather(tbuf, [r + 2 * TCOLS])
            tlo = plsc.load_gather(tbuf, [r + 3 * TCOLS])
            thi = plsc.load_gather(tbuf, [r + 4 * TCOLS])
            vlo = plsc.load_gather(tbuf, [r + 5 * TCOLS])
            vhi = plsc.load_gather(tbuf, [r + 6 * TCOLS])
            y = a + b * x
            y = jnp.where(x >= thi, vhi, y)
            y = jnp.where(x <= tlo, vlo, y)
            xbuf[pl.ds(s0, LANES)] = y
        return _

    lax.fori_loop(0, GROUPS, gbody, None)
    pltpu.sync_copy(xbuf, out_hbm.at[pl.ds(base, CHUNK)])


def _sc_map(src_f, tab_f):
    mesh = plsc.VectorSubcoreMesh(
        core_axis_name="c", subcore_axis_name="s", num_cores=NC,
        num_subcores=NS)
    return pl.kernel(
        _map_body,
        out_type=jax.ShapeDtypeStruct((NTOT,), jnp.float32),
        mesh=mesh,
        compiler_params=pltpu.CompilerParams(needs_layout_passes=False),
        scratch_types=[
            pltpu.VMEM((CHUNK,), jnp.float32),
            pltpu.VMEM((TROWS * TCOLS,), jnp.float32),
        ],
    )(src_f, tab_f)


def kernel(src, tgt):
    src_f = src.reshape(-1)
    tgt_f = tgt.reshape(-1)
    hparts = _sc_hist(src_f, tgt_f).reshape(NW, NHIST)
    tab = _tc_tables(hparts)
    out_f = _sc_map(src_f, tab.reshape(-1))
    return out_f.reshape(H, W, C)
